# baseline (speedup 1.0000x reference)
import jax
import jax.numpy as jnp
from jax import lax
from jax.experimental import pallas as pl
from jax.experimental.pallas import tpu as pltpu
from jax.experimental.pallas import tpu_sc as plsc

BATCH = 16384
EMBED_DIM = 128
CHUNK = 128
NCHUNK = 4
MLP_BLK = 2048

_NC, _NS = 2, 16
_NW = _NC * _NS
_NSLOT = 4


def _sc_gather_body(n_rows, users_hbm, items_hbm, ut_hbm, it_hbm, x_hbm,
                    idx_v, rows_v, gsem, ssem):
    ch_per_w = n_rows // (_NW * CHUNK)
    ntask = 2 * ch_per_w
    wid = lax.axis_index("s") * _NC + lax.axis_index("c")
    r0 = wid * ch_per_w
    pltpu.sync_copy(users_hbm.at[pl.ds(r0, ch_per_w)],
                    idx_v.at[pl.ds(0, ch_per_w)])
    pltpu.sync_copy(items_hbm.at[pl.ds(r0, ch_per_w)],
                    idx_v.at[pl.ds(ch_per_w, ch_per_w)])

    def dst(k):
        col = k // ch_per_w
        j = k % ch_per_w
        return x_hbm.at[pl.ds((r0 + j) * CHUNK, CHUNK),
                        pl.ds(col * EMBED_DIM, EMBED_DIM)]

    gd, sd = {}, {}
    for k in range(ntask):
        s = k % _NSLOT
        tbl = ut_hbm if k < ch_per_w else it_hbm
        if k >= _NSLOT:
            sd[k - _NSLOT].wait()
        gd[k] = pltpu.async_copy(tbl.at[idx_v.at[k]], rows_v.at[s], gsem.at[s])
        if k >= 1:
            gd[k - 1].wait()
            sd[k - 1] = pltpu.async_copy(rows_v.at[(k - 1) % _NSLOT],
                                         dst(k - 1), ssem.at[(k - 1) % _NSLOT])
    gd[ntask - 1].wait()
    sd[ntask - 1] = pltpu.async_copy(rows_v.at[(ntask - 1) % _NSLOT],
                                     dst(ntask - 1),
                                     ssem.at[(ntask - 1) % _NSLOT])
    for k in range(max(0, ntask - _NSLOT), ntask):
        sd[k].wait()


def _sc_gather(users_r, items_r, user_table, item_table):
    n_rows = users_r.shape[0] * CHUNK
    ntask = 2 * (n_rows // (_NW * CHUNK))
    mesh = plsc.VectorSubcoreMesh(core_axis_name="c", subcore_axis_name="s")
    f = pl.kernel(
        lambda *a: _sc_gather_body(n_rows, *a),
        mesh=mesh,
        out_type=jax.ShapeDtypeStruct((n_rows, 2 * EMBED_DIM), jnp.float32),
        scratch_types=[
            pltpu.VMEM((ntask, CHUNK), jnp.int32),
            pltpu.VMEM((_NSLOT, CHUNK, EMBED_DIM), jnp.float32),
            pltpu.SemaphoreType.DMA((_NSLOT,)),
            pltpu.SemaphoreType.DMA((_NSLOT,)),
        ],
    )
    return f(users_r, items_r, user_table, item_table)


def _mlp_body(x_ref, w0_ref, b0_ref, w1_ref, b1_ref, w2_ref, b2_ref, out_ref):
    bf = jnp.bfloat16
    h = jnp.dot(x_ref[...].astype(bf), w0_ref[...],
                preferred_element_type=jnp.float32)
    h = jnp.maximum(h + b0_ref[...], 0.0)
    h = jnp.dot(h.astype(bf), w1_ref[...], preferred_element_type=jnp.float32)
    h = jnp.maximum(h + b1_ref[...], 0.0)
    h = jnp.dot(h.astype(bf), w2_ref[...], preferred_element_type=jnp.float32)
    out_ref[...] = jnp.maximum(h + b2_ref[...], 0.0)


def _mlp_chunk(x_c, weights, prev, c):
    W0, b0, W1, b1, W2, b2 = weights
    rows = x_c.shape[0]
    n_blk = rows // MLP_BLK
    h0, h1, h2 = W0.shape[1], W1.shape[1], W2.shape[1]
    data_specs = [
        pl.BlockSpec((MLP_BLK, 2 * EMBED_DIM), lambda i: (i, 0)),
        pl.BlockSpec((2 * EMBED_DIM, h0), lambda i: (0, 0)),
        pl.BlockSpec((1, h0), lambda i: (0, 0)),
        pl.BlockSpec((h0, h1), lambda i: (0, 0)),
        pl.BlockSpec((1, h1), lambda i: (0, 0)),
        pl.BlockSpec((h1, h2), lambda i: (0, 0)),
        pl.BlockSpec((1, h2), lambda i: (0, 0)),
    ]
    args = (x_c, W0, b0, W1, b1, W2, b2)
    if prev is None:
        body = _mlp_body
        in_specs = data_specs
        aliases = {}
    else:
        def body(prev_ref, *refs):
            _mlp_body(*refs)
        in_specs = [pl.BlockSpec(memory_space=pltpu.MemorySpace.HBM)] + data_specs
        args = (prev,) + args
        aliases = {0: 0}
    return pl.pallas_call(
        body,
        grid=(n_blk,),
        in_specs=in_specs,
        out_specs=pl.BlockSpec((MLP_BLK, h2), lambda i, c=c: (i + c * n_blk, 0)),
        out_shape=jax.ShapeDtypeStruct((BATCH, h2), jnp.float32),
        input_output_aliases=aliases,
        compiler_params=pltpu.CompilerParams(
            dimension_semantics=("arbitrary",),
        ),
    )(*args)


def kernel(users, items, user_table, item_table, W0, b0, W1, b1, W2, b2):
    h0, h1, h2 = W0.shape[1], W1.shape[1], W2.shape[1]
    weights = (W0.astype(jnp.bfloat16), b0.reshape(1, h0),
               W1.astype(jnp.bfloat16), b1.reshape(1, h1),
               W2.astype(jnp.bfloat16), b2.reshape(1, h2))
    users_r = users.astype(jnp.int32).reshape(BATCH // CHUNK, CHUNK)
    items_r = items.astype(jnp.int32).reshape(BATCH // CHUNK, CHUNK)
    rows_per_chunk = BATCH // NCHUNK
    ir_per_chunk = rows_per_chunk // CHUNK
    xs = []
    for c in range(NCHUNK):
        xs.append(_sc_gather(users_r[c * ir_per_chunk:(c + 1) * ir_per_chunk],
                             items_r[c * ir_per_chunk:(c + 1) * ir_per_chunk],
                             user_table, item_table))
    out = None
    for c in range(NCHUNK):
        out = _mlp_chunk(xs[c], weights, out, c)
    return out


# speedup vs baseline: 1.0988x; 1.0988x over previous
import jax
import jax.numpy as jnp
from jax import lax
from jax.experimental import pallas as pl
from jax.experimental.pallas import tpu as pltpu
from jax.experimental.pallas import tpu_sc as plsc

BATCH = 16384
EMBED_DIM = 128
CHUNK = 128
NCHUNK = 2
MLP_BLK = 2048

_NC, _NS = 2, 16
_NW = _NC * _NS
_NSLOT = 4


def _sc_gather_body(n_rows, base_ir, users_hbm, items_hbm, ut_hbm, it_hbm,
                    x_hbm, idx_v, rows_v, gsem, ssem):
    ch_per_w = n_rows // (_NW * CHUNK)
    ntask = 2 * ch_per_w
    wid = lax.axis_index("s") * _NC + lax.axis_index("c")
    r0 = wid * ch_per_w
    pltpu.sync_copy(users_hbm.at[pl.ds(base_ir + r0, ch_per_w)],
                    idx_v.at[pl.ds(0, ch_per_w)])
    pltpu.sync_copy(items_hbm.at[pl.ds(base_ir + r0, ch_per_w)],
                    idx_v.at[pl.ds(ch_per_w, ch_per_w)])

    def dst(k):
        col = k // ch_per_w
        j = k % ch_per_w
        return x_hbm.at[pl.ds((r0 + j) * CHUNK, CHUNK),
                        pl.ds(col * EMBED_DIM, EMBED_DIM)]

    gd, sd = {}, {}
    for k in range(ntask):
        s = k % _NSLOT
        tbl = ut_hbm if k < ch_per_w else it_hbm
        if k >= _NSLOT:
            sd[k - _NSLOT].wait()
        gd[k] = pltpu.async_copy(tbl.at[idx_v.at[k]], rows_v.at[s], gsem.at[s])
        if k >= 1:
            gd[k - 1].wait()
            sd[k - 1] = pltpu.async_copy(rows_v.at[(k - 1) % _NSLOT],
                                         dst(k - 1), ssem.at[(k - 1) % _NSLOT])
    gd[ntask - 1].wait()
    sd[ntask - 1] = pltpu.async_copy(rows_v.at[(ntask - 1) % _NSLOT],
                                     dst(ntask - 1),
                                     ssem.at[(ntask - 1) % _NSLOT])
    for k in range(max(0, ntask - _NSLOT), ntask):
        sd[k].wait()


def _sc_gather(users_r, items_r, user_table, item_table, n_rows, base_ir):
    ntask = 2 * (n_rows // (_NW * CHUNK))
    mesh = plsc.VectorSubcoreMesh(core_axis_name="c", subcore_axis_name="s")
    f = pl.kernel(
        lambda *a: _sc_gather_body(n_rows, base_ir, *a),
        mesh=mesh,
        out_type=jax.ShapeDtypeStruct((n_rows, 2 * EMBED_DIM), jnp.float32),
        scratch_types=[
            pltpu.VMEM((ntask, CHUNK), jnp.int32),
            pltpu.VMEM((_NSLOT, CHUNK, EMBED_DIM), jnp.float32),
            pltpu.SemaphoreType.DMA((_NSLOT,)),
            pltpu.SemaphoreType.DMA((_NSLOT,)),
        ],
    )
    return f(users_r, items_r, user_table, item_table)


def _mlp_body(x_ref, w0_ref, b0_ref, w1_ref, b1_ref, w2_ref, b2_ref, out_ref):
    bf = jnp.bfloat16
    h = jnp.dot(x_ref[...].astype(bf), w0_ref[...],
                preferred_element_type=jnp.float32)
    h = jnp.maximum(h + b0_ref[...], 0.0)
    h = jnp.dot(h.astype(bf), w1_ref[...], preferred_element_type=jnp.float32)
    h = jnp.maximum(h + b1_ref[...], 0.0)
    h = jnp.dot(h.astype(bf), w2_ref[...], preferred_element_type=jnp.float32)
    out_ref[...] = jnp.maximum(h + b2_ref[...], 0.0)


def _mlp_chunk(x_c, weights, prev, c):
    W0, b0, W1, b1, W2, b2 = weights
    rows = x_c.shape[0]
    n_blk = rows // MLP_BLK
    h0, h1, h2 = W0.shape[1], W1.shape[1], W2.shape[1]
    data_specs = [
        pl.BlockSpec((MLP_BLK, 2 * EMBED_DIM), lambda i: (i, 0)),
        pl.BlockSpec((2 * EMBED_DIM, h0), lambda i: (0, 0)),
        pl.BlockSpec((1, h0), lambda i: (0, 0)),
        pl.BlockSpec((h0, h1), lambda i: (0, 0)),
        pl.BlockSpec((1, h1), lambda i: (0, 0)),
        pl.BlockSpec((h1, h2), lambda i: (0, 0)),
        pl.BlockSpec((1, h2), lambda i: (0, 0)),
    ]
    args = (x_c, W0, b0, W1, b1, W2, b2)
    if prev is None:
        body = _mlp_body
        in_specs = data_specs
        aliases = {}
    else:
        def body(prev_ref, *refs):
            _mlp_body(*refs)
        in_specs = [pl.BlockSpec(memory_space=pltpu.MemorySpace.HBM)] + data_specs
        args = (prev,) + args
        aliases = {0: 0}
    return pl.pallas_call(
        body,
        grid=(n_blk,),
        in_specs=in_specs,
        out_specs=pl.BlockSpec((MLP_BLK, h2), lambda i, c=c: (i + c * n_blk, 0)),
        out_shape=jax.ShapeDtypeStruct((BATCH, h2), jnp.float32),
        input_output_aliases=aliases,
        compiler_params=pltpu.CompilerParams(
            dimension_semantics=("arbitrary",),
        ),
    )(*args)


def kernel(users, items, user_table, item_table, W0, b0, W1, b1, W2, b2):
    h0, h1, h2 = W0.shape[1], W1.shape[1], W2.shape[1]
    weights = (W0.astype(jnp.bfloat16), b0.reshape(1, h0),
               W1.astype(jnp.bfloat16), b1.reshape(1, h1),
               W2.astype(jnp.bfloat16), b2.reshape(1, h2))
    users_r = users.astype(jnp.int32).reshape(BATCH // CHUNK, CHUNK)
    items_r = items.astype(jnp.int32).reshape(BATCH // CHUNK, CHUNK)
    rows_per_chunk = BATCH // NCHUNK
    ir_per_chunk = rows_per_chunk // CHUNK
    xs = []
    for c in range(NCHUNK):
        xs.append(_sc_gather(users_r, items_r, user_table, item_table,
                             rows_per_chunk, c * ir_per_chunk))
    out = None
    for c in range(NCHUNK):
        out = _mlp_chunk(xs[c], weights, out, c)
    return out
